# Initial kernel scaffold; baseline (speedup 1.0000x reference)
import jax
import jax.numpy as jnp
from jax.experimental import pallas as pl


def kernel(inputs, targets, features):
    raise NotImplementedError("write your pallas kernel here")


# speedup vs baseline: 1.0088x; 1.0088x over previous
import functools

import jax
import jax.numpy as jnp
from jax import lax
from jax.experimental import pallas as pl
from jax.experimental.pallas import tpu as pltpu
from jax.experimental.pallas import tpu_sc as plsc

_NF = 32
_NCLS = 100000
_B = 1024
_TEMP = 0.05
_BK = 2048
_GRID = (_NCLS + _BK - 1) // _BK
_NPAD = _GRID * _BK


def _tc_body(x_ref, f_ref, g_ref, out_ref, m_ref, s_ref):
    pid = pl.program_id(0)

    @pl.when(pid == 0)
    def _init():
        m_ref[...] = jnp.full((_B, 1), -1e30, jnp.float32)
        s_ref[...] = jnp.zeros((_B, 1), jnp.float32)

    x = x_ref[...]
    blk = lax.dot_general(
        x, f_ref[...], (((1,), (1,)), ((), ())),
        preferred_element_type=jnp.float32,
        precision=lax.Precision.HIGHEST,
    ) * (1.0 / _TEMP)
    col = pid * _BK + lax.broadcasted_iota(jnp.int32, (1, _BK), 1)
    blk = jnp.where(col < _NCLS, blk, -1e30)

    bm = jnp.max(blk, axis=1, keepdims=True)
    m_old = m_ref[...]
    m_new = jnp.maximum(m_old, bm)
    p = jnp.exp(blk - m_new)
    s_ref[...] = s_ref[...] * jnp.exp(m_old - m_new) + jnp.sum(
        p, axis=1, keepdims=True)
    m_ref[...] = m_new

    @pl.when(pid == _GRID - 1)
    def _fin():
        lse = m_ref[...] + jnp.log(s_ref[...])
        tgt = jnp.sum(x * g_ref[...], axis=1, keepdims=True) * (1.0 / _TEMP)
        loss = jnp.sum(lse - tgt) * (1.0 / _B)
        out_ref[...] = jnp.full((8, 128), loss, jnp.float32)


def _lse_loss(inputs, fpad, gathered):
    return pl.pallas_call(
        _tc_body,
        grid=(_GRID,),
        in_specs=[
            pl.BlockSpec((_B, _NF), lambda i: (0, 0)),
            pl.BlockSpec((_BK, _NF), lambda i: (i, 0)),
            pl.BlockSpec((_B, _NF), lambda i: (0, 0)),
        ],
        out_specs=pl.BlockSpec((8, 128), lambda i: (0, 0)),
        out_shape=jax.ShapeDtypeStruct((8, 128), jnp.float32),
        scratch_shapes=[
            pltpu.VMEM((_B, 1), jnp.float32),
            pltpu.VMEM((_B, 1), jnp.float32),
        ],
        compiler_params=pltpu.CompilerParams(
            dimension_semantics=("arbitrary",)),
    )(inputs, fpad, gathered)


@functools.cache
def _make_sc_gather():
    info = plsc.get_sparse_core_info()
    nc, ns = info.num_cores, info.num_subcores
    nw = nc * ns
    b_per_w = _B // nw
    mesh = plsc.VectorSubcoreMesh(core_axis_name="c", subcore_axis_name="s")

    @functools.partial(
        pl.kernel, mesh=mesh,
        out_type=jax.ShapeDtypeStruct((_B, _NF), jnp.float32),
        scratch_types=[
            pltpu.VMEM((b_per_w,), jnp.int32),
            pltpu.VMEM((b_per_w, _NF), jnp.float32),
            pltpu.SemaphoreType.DMA,
        ],
        compiler_params=pltpu.CompilerParams(use_tc_tiling_on_sc=False),
    )
    def gather(table_hbm, idx_hbm, out_hbm, idx_v, rows_v, sem):
        wid = lax.axis_index("s") * nc + lax.axis_index("c")
        base = wid * b_per_w
        pltpu.sync_copy(idx_hbm.at[pl.ds(base, b_per_w)], idx_v)
        pltpu.async_copy(table_hbm.at[idx_v], rows_v, sem).wait()
        pltpu.sync_copy(rows_v, out_hbm.at[pl.ds(base, b_per_w)])

    return gather


def kernel(inputs, targets, features):
    idx = targets.astype(jnp.int32)
    gathered = _make_sc_gather()(features, idx)
    fpad = jnp.pad(features, ((0, _NPAD - _NCLS), (0, 0)))
    out = _lse_loss(inputs, fpad, gathered)
    return out[0, 0]


# speedup vs baseline: 2.0030x; 1.9855x over previous
import functools

import jax
import jax.numpy as jnp
from jax import lax
from jax.experimental import pallas as pl
from jax.experimental.pallas import tpu as pltpu
from jax.experimental.pallas import tpu_sc as plsc

_NF = 32
_NCLS = 100000
_B = 1024
_TEMP = 0.05
_BK = 2048
_GRID = (_NCLS + _BK - 1) // _BK
_NPAD = _GRID * _BK


def _tc_body(x_ref, f_ref, g_ref, out_ref, m_ref, s_ref):
    pid = pl.program_id(0)

    @pl.when(pid == 0)
    def _init():
        m_ref[...] = jnp.full((_B, 1), -1e30, jnp.float32)
        s_ref[...] = jnp.zeros((_B, 1), jnp.float32)

    x = x_ref[...]
    blk = lax.dot_general(
        x, f_ref[...], (((1,), (1,)), ((), ())),
        preferred_element_type=jnp.float32,
        precision=lax.Precision.DEFAULT,
    ) * (1.0 / _TEMP)
    col = pid * _BK + lax.broadcasted_iota(jnp.int32, (1, _BK), 1)
    blk = jnp.where(col < _NCLS, blk, -1e30)

    bm = jnp.max(blk, axis=1, keepdims=True)
    m_old = m_ref[...]
    m_new = jnp.maximum(m_old, bm)
    p = jnp.exp(blk - m_new)
    s_ref[...] = s_ref[...] * jnp.exp(m_old - m_new) + jnp.sum(
        p, axis=1, keepdims=True)
    m_ref[...] = m_new

    @pl.when(pid == _GRID - 1)
    def _fin():
        lse = m_ref[...] + jnp.log(s_ref[...])
        tgt = jnp.sum(x * g_ref[...], axis=1, keepdims=True) * (1.0 / _TEMP)
        loss = jnp.sum(lse - tgt) * (1.0 / _B)
        out_ref[...] = jnp.full((8, 128), loss, jnp.float32)


def _lse_loss(inputs, fpad, gathered):
    return pl.pallas_call(
        _tc_body,
        grid=(_GRID,),
        in_specs=[
            pl.BlockSpec((_B, _NF), lambda i: (0, 0)),
            pl.BlockSpec((_BK, _NF), lambda i: (i, 0)),
            pl.BlockSpec((_B, _NF), lambda i: (0, 0)),
        ],
        out_specs=pl.BlockSpec((8, 128), lambda i: (0, 0)),
        out_shape=jax.ShapeDtypeStruct((8, 128), jnp.float32),
        scratch_shapes=[
            pltpu.VMEM((_B, 1), jnp.float32),
            pltpu.VMEM((_B, 1), jnp.float32),
        ],
        compiler_params=pltpu.CompilerParams(
            dimension_semantics=("arbitrary",)),
    )(inputs, fpad, gathered)


@functools.cache
def _make_sc_gather():
    info = plsc.get_sparse_core_info()
    nc, ns = info.num_cores, info.num_subcores
    nw = nc * ns
    b_per_w = _B // nw
    mesh = plsc.VectorSubcoreMesh(core_axis_name="c", subcore_axis_name="s")

    @functools.partial(
        pl.kernel, mesh=mesh,
        out_type=jax.ShapeDtypeStruct((_B, _NF), jnp.float32),
        scratch_types=[
            pltpu.VMEM((b_per_w,), jnp.int32),
            pltpu.VMEM((b_per_w, _NF), jnp.float32),
            pltpu.SemaphoreType.DMA,
        ],
        compiler_params=pltpu.CompilerParams(use_tc_tiling_on_sc=False),
    )
    def gather(table_hbm, idx_hbm, out_hbm, idx_v, rows_v, sem):
        wid = lax.axis_index("s") * nc + lax.axis_index("c")
        base = wid * b_per_w
        pltpu.sync_copy(idx_hbm.at[pl.ds(base, b_per_w)], idx_v)
        pltpu.async_copy(table_hbm.at[idx_v], rows_v, sem).wait()
        pltpu.sync_copy(rows_v, out_hbm.at[pl.ds(base, b_per_w)])

    return gather


def kernel(inputs, targets, features):
    idx = targets.astype(jnp.int32)
    gathered = _make_sc_gather()(features, idx)
    fpad = jnp.pad(features, ((0, _NPAD - _NCLS), (0, 0)))
    out = _lse_loss(inputs, fpad, gathered)
    return out[0, 0]


# speedup vs baseline: 2.2967x; 1.1466x over previous
import functools

import jax
import jax.numpy as jnp
from jax import lax
from jax.experimental import pallas as pl
from jax.experimental.pallas import tpu as pltpu
from jax.experimental.pallas import tpu_sc as plsc

_NF = 32
_NCLS = 100000
_B = 1024
_TEMP = 0.05
_BK = 2048
_GRID = (_NCLS + _BK - 1) // _BK
_NPAD = _GRID * _BK


_LOG2E = 1.4426950408889634
_SCALE = _LOG2E / _TEMP
_LN2 = 0.6931471805599453


def _tc_body(x_ref, f_ref, g_ref, out_ref, m_ref, s_ref):
    pid = pl.program_id(0)

    @pl.when(pid == 0)
    def _init():
        m_ref[...] = jnp.full((_B, 1), -1e30, jnp.float32)
        s_ref[...] = jnp.zeros((_B, 1), jnp.float32)

    xs = x_ref[...] * _SCALE
    blk = lax.dot_general(
        xs, f_ref[...], (((1,), (1,)), ((), ())),
        preferred_element_type=jnp.float32,
        precision=lax.Precision.DEFAULT,
    )

    def _update(b):
        bm = jnp.max(b, axis=1, keepdims=True)
        m_old = m_ref[...]
        m_new = jnp.maximum(m_old, bm)
        p = jnp.exp2(b - m_new)
        s_ref[...] = s_ref[...] * jnp.exp2(m_old - m_new) + jnp.sum(
            p, axis=1, keepdims=True)
        m_ref[...] = m_new

    @pl.when(pid != _GRID - 1)
    def _full():
        _update(blk)

    @pl.when(pid == _GRID - 1)
    def _tail():
        col = lax.broadcasted_iota(jnp.int32, (1, _BK), 1)
        _update(jnp.where(col < _NCLS - (_GRID - 1) * _BK, blk, -1e30))
        lse2 = m_ref[...] + jnp.log2(s_ref[...])
        tgt2 = jnp.sum(xs * g_ref[...], axis=1, keepdims=True)
        loss = jnp.sum(lse2 - tgt2) * (_LN2 / _B)
        out_ref[...] = jnp.full((8, 128), loss, jnp.float32)


def _lse_loss(inputs, fpad, gathered):
    return pl.pallas_call(
        _tc_body,
        grid=(_GRID,),
        in_specs=[
            pl.BlockSpec((_B, _NF), lambda i: (0, 0)),
            pl.BlockSpec((_BK, _NF), lambda i: (i, 0)),
            pl.BlockSpec((_B, _NF), lambda i: (0, 0)),
        ],
        out_specs=pl.BlockSpec((8, 128), lambda i: (0, 0)),
        out_shape=jax.ShapeDtypeStruct((8, 128), jnp.float32),
        scratch_shapes=[
            pltpu.VMEM((_B, 1), jnp.float32),
            pltpu.VMEM((_B, 1), jnp.float32),
        ],
        compiler_params=pltpu.CompilerParams(
            dimension_semantics=("arbitrary",)),
    )(inputs, fpad, gathered)


@functools.cache
def _make_sc_gather():
    info = plsc.get_sparse_core_info()
    nc, ns = info.num_cores, info.num_subcores
    nw = nc * ns
    b_per_w = _B // nw
    mesh = plsc.VectorSubcoreMesh(core_axis_name="c", subcore_axis_name="s")

    @functools.partial(
        pl.kernel, mesh=mesh,
        out_type=jax.ShapeDtypeStruct((_B, _NF), jnp.float32),
        scratch_types=[
            pltpu.VMEM((b_per_w,), jnp.int32),
            pltpu.VMEM((b_per_w, _NF), jnp.float32),
            pltpu.SemaphoreType.DMA,
        ],
        compiler_params=pltpu.CompilerParams(use_tc_tiling_on_sc=False),
    )
    def gather(table_hbm, idx_hbm, out_hbm, idx_v, rows_v, sem):
        wid = lax.axis_index("s") * nc + lax.axis_index("c")
        base = wid * b_per_w
        pltpu.sync_copy(idx_hbm.at[pl.ds(base, b_per_w)], idx_v)
        pltpu.async_copy(table_hbm.at[idx_v], rows_v, sem).wait()
        pltpu.sync_copy(rows_v, out_hbm.at[pl.ds(base, b_per_w)])

    return gather


def kernel(inputs, targets, features):
    idx = targets.astype(jnp.int32)
    gathered = _make_sc_gather()(features, idx)
    out = _lse_loss(inputs, features, gathered)
    return out[0, 0]


# speedup vs baseline: 2.3711x; 1.0324x over previous
import functools

import jax
import jax.numpy as jnp
from jax import lax
from jax.experimental import pallas as pl
from jax.experimental.pallas import tpu as pltpu
from jax.experimental.pallas import tpu_sc as plsc

_NF = 32
_NCLS = 100000
_B = 1024
_TEMP = 0.05
_BK = 4096
_GRID = (_NCLS + _BK - 1) // _BK
_NPAD = _GRID * _BK


_LOG2E = 1.4426950408889634
_SCALE = _LOG2E / _TEMP
_LN2 = 0.6931471805599453


def _tc_body(x_ref, f_ref, g_ref, out_ref, m_ref, s_ref):
    pid = pl.program_id(0)

    @pl.when(pid == 0)
    def _init():
        m_ref[...] = jnp.full((_B, 1), -1e30, jnp.float32)
        s_ref[...] = jnp.zeros((_B, 1), jnp.float32)

    xs = x_ref[...] * _SCALE
    blk = lax.dot_general(
        xs, f_ref[...], (((1,), (1,)), ((), ())),
        preferred_element_type=jnp.float32,
        precision=lax.Precision.DEFAULT,
    )

    def _update(b):
        bm = jnp.max(b, axis=1, keepdims=True)
        m_old = m_ref[...]
        m_new = jnp.maximum(m_old, bm)
        p = jnp.exp2(b - m_new)
        s_ref[...] = s_ref[...] * jnp.exp2(m_old - m_new) + jnp.sum(
            p, axis=1, keepdims=True)
        m_ref[...] = m_new

    @pl.when(pid != _GRID - 1)
    def _full():
        _update(blk)

    @pl.when(pid == _GRID - 1)
    def _tail():
        col = lax.broadcasted_iota(jnp.int32, (1, _BK), 1)
        _update(jnp.where(col < _NCLS - (_GRID - 1) * _BK, blk, -1e30))
        lse2 = m_ref[...] + jnp.log2(s_ref[...])
        tgt2 = jnp.sum(xs * g_ref[...], axis=1, keepdims=True)
        loss = jnp.sum(lse2 - tgt2) * (_LN2 / _B)
        out_ref[...] = jnp.full((8, 128), loss, jnp.float32)


def _lse_loss(inputs, fpad, gathered):
    return pl.pallas_call(
        _tc_body,
        grid=(_GRID,),
        in_specs=[
            pl.BlockSpec((_B, _NF), lambda i: (0, 0)),
            pl.BlockSpec((_BK, _NF), lambda i: (i, 0)),
            pl.BlockSpec((_B, _NF), lambda i: (0, 0)),
        ],
        out_specs=pl.BlockSpec((8, 128), lambda i: (0, 0)),
        out_shape=jax.ShapeDtypeStruct((8, 128), jnp.float32),
        scratch_shapes=[
            pltpu.VMEM((_B, 1), jnp.float32),
            pltpu.VMEM((_B, 1), jnp.float32),
        ],
        compiler_params=pltpu.CompilerParams(
            dimension_semantics=("arbitrary",)),
    )(inputs, fpad, gathered)


@functools.cache
def _make_sc_gather():
    info = plsc.get_sparse_core_info()
    nc, ns = info.num_cores, info.num_subcores
    nw = nc * ns
    b_per_w = _B // nw
    mesh = plsc.VectorSubcoreMesh(core_axis_name="c", subcore_axis_name="s")

    @functools.partial(
        pl.kernel, mesh=mesh,
        out_type=jax.ShapeDtypeStruct((_B, _NF), jnp.float32),
        scratch_types=[
            pltpu.VMEM((b_per_w,), jnp.int32),
            pltpu.VMEM((b_per_w, _NF), jnp.float32),
            pltpu.SemaphoreType.DMA,
        ],
        compiler_params=pltpu.CompilerParams(use_tc_tiling_on_sc=False),
    )
    def gather(table_hbm, idx_hbm, out_hbm, idx_v, rows_v, sem):
        wid = lax.axis_index("s") * nc + lax.axis_index("c")
        base = wid * b_per_w
        pltpu.sync_copy(idx_hbm.at[pl.ds(base, b_per_w)], idx_v)
        pltpu.async_copy(table_hbm.at[idx_v], rows_v, sem).wait()
        pltpu.sync_copy(rows_v, out_hbm.at[pl.ds(base, b_per_w)])

    return gather


def kernel(inputs, targets, features):
    idx = targets.astype(jnp.int32)
    gathered = _make_sc_gather()(features, idx)
    out = _lse_loss(inputs, features, gathered)
    return out[0, 0]


# speedup vs baseline: 2.5523x; 1.0764x over previous
import functools

import jax
import jax.numpy as jnp
from jax import lax
from jax.experimental import pallas as pl
from jax.experimental.pallas import tpu as pltpu
from jax.experimental.pallas import tpu_sc as plsc

_NF = 32
_NCLS = 100000
_B = 1024
_TEMP = 0.05
_BK = 4096
_GRID = (_NCLS + _BK - 1) // _BK
_NPAD = _GRID * _BK


_LOG2E = 1.4426950408889634
_SCALE = _LOG2E / _TEMP
_LN2 = 0.6931471805599453


def _tc_body(x_ref, f_ref, g_ref, out_ref, m_ref, s_ref):
    pid = pl.program_id(0)

    @pl.when(pid == 0)
    def _init():
        m_ref[...] = jnp.full((_B, 1), -1e30, jnp.float32)
        s_ref[...] = jnp.zeros((_B, 1), jnp.float32)

    xs = x_ref[...] * _SCALE

    def _dot():
        return lax.dot_general(
            xs, f_ref[...], (((1,), (1,)), ((), ())),
            preferred_element_type=jnp.float32,
            precision=lax.Precision.DEFAULT,
        )

    def _update(b):
        bm = jnp.max(b, axis=1, keepdims=True)
        m_old = m_ref[...]
        m_new = jnp.maximum(m_old, bm)
        p = jnp.exp2(b - m_new)
        s_ref[...] = s_ref[...] * jnp.exp2(m_old - m_new) + jnp.sum(
            p, axis=1, keepdims=True)
        m_ref[...] = m_new

    @pl.when(pid != _GRID - 1)
    def _full():
        _update(_dot())

    @pl.when(pid == _GRID - 1)
    def _tail():
        col = lax.broadcasted_iota(jnp.int32, (1, _BK), 1)
        _update(jnp.where(col < _NCLS - (_GRID - 1) * _BK, _dot(), -1e30))
        lse2 = m_ref[...] + jnp.log2(s_ref[...])
        tgt2 = jnp.sum(xs * g_ref[...], axis=1, keepdims=True)
        loss = jnp.sum(lse2 - tgt2) * (_LN2 / _B)
        out_ref[...] = jnp.full((8, 128), loss, jnp.float32)


def _lse_loss(inputs, fpad, gathered):
    return pl.pallas_call(
        _tc_body,
        grid=(_GRID,),
        in_specs=[
            pl.BlockSpec((_B, _NF), lambda i: (0, 0)),
            pl.BlockSpec((_BK, _NF), lambda i: (i, 0)),
            pl.BlockSpec((_B, _NF), lambda i: (0, 0)),
        ],
        out_specs=pl.BlockSpec((8, 128), lambda i: (0, 0)),
        out_shape=jax.ShapeDtypeStruct((8, 128), jnp.float32),
        scratch_shapes=[
            pltpu.VMEM((_B, 1), jnp.float32),
            pltpu.VMEM((_B, 1), jnp.float32),
        ],
        compiler_params=pltpu.CompilerParams(
            dimension_semantics=("arbitrary",)),
    )(inputs, fpad, gathered)


@functools.cache
def _make_sc_gather():
    info = plsc.get_sparse_core_info()
    nc, ns = info.num_cores, info.num_subcores
    nw = nc * ns
    b_per_w = _B // nw
    mesh = plsc.VectorSubcoreMesh(core_axis_name="c", subcore_axis_name="s")

    @functools.partial(
        pl.kernel, mesh=mesh,
        out_type=jax.ShapeDtypeStruct((_B, _NF), jnp.float32),
        scratch_types=[
            pltpu.VMEM((b_per_w,), jnp.int32),
            pltpu.VMEM((b_per_w, _NF), jnp.float32),
            pltpu.SemaphoreType.DMA,
        ],
        compiler_params=pltpu.CompilerParams(use_tc_tiling_on_sc=False),
    )
    def gather(table_hbm, idx_hbm, out_hbm, idx_v, rows_v, sem):
        wid = lax.axis_index("s") * nc + lax.axis_index("c")
        base = wid * b_per_w
        pltpu.sync_copy(idx_hbm.at[pl.ds(base, b_per_w)], idx_v)
        pltpu.async_copy(table_hbm.at[idx_v], rows_v, sem).wait()
        pltpu.sync_copy(rows_v, out_hbm.at[pl.ds(base, b_per_w)])

    return gather


def kernel(inputs, targets, features):
    idx = targets.astype(jnp.int32)
    gathered = _make_sc_gather()(features, idx)
    out = _lse_loss(inputs, features, gathered)
    return out[0, 0]


# speedup vs baseline: 2.6151x; 1.0246x over previous
import functools

import jax
import jax.numpy as jnp
from jax import lax
from jax.experimental import pallas as pl
from jax.experimental.pallas import tpu as pltpu
from jax.experimental.pallas import tpu_sc as plsc

_NF = 32
_NCLS = 100000
_B = 1024
_TEMP = 0.05
_BK = 8192
_GRID = (_NCLS + _BK - 1) // _BK
_NPAD = _GRID * _BK


_LOG2E = 1.4426950408889634
_SCALE = _LOG2E / _TEMP
_LN2 = 0.6931471805599453


def _tc_body(x_ref, f_ref, g_ref, out_ref, m_ref, s_ref):
    pid = pl.program_id(0)

    @pl.when(pid == 0)
    def _init():
        m_ref[...] = jnp.full((_B, 1), -1e30, jnp.float32)
        s_ref[...] = jnp.zeros((_B, 1), jnp.float32)

    xs = x_ref[...] * _SCALE

    def _dot():
        return lax.dot_general(
            xs, f_ref[...], (((1,), (1,)), ((), ())),
            preferred_element_type=jnp.float32,
            precision=lax.Precision.DEFAULT,
        )

    def _update(b):
        bm = jnp.max(b, axis=1, keepdims=True)
        m_old = m_ref[...]
        m_new = jnp.maximum(m_old, bm)
        p = jnp.exp2(b - m_new)
        s_ref[...] = s_ref[...] * jnp.exp2(m_old - m_new) + jnp.sum(
            p, axis=1, keepdims=True)
        m_ref[...] = m_new

    @pl.when(pid != _GRID - 1)
    def _full():
        _update(_dot())

    @pl.when(pid == _GRID - 1)
    def _tail():
        col = lax.broadcasted_iota(jnp.int32, (1, _BK), 1)
        _update(jnp.where(col < _NCLS - (_GRID - 1) * _BK, _dot(), -1e30))
        lse2 = m_ref[...] + jnp.log2(s_ref[...])
        tgt2 = jnp.sum(xs * g_ref[...], axis=1, keepdims=True)
        loss = jnp.sum(lse2 - tgt2) * (_LN2 / _B)
        out_ref[...] = jnp.full((8, 128), loss, jnp.float32)


def _lse_loss(inputs, fpad, gathered):
    return pl.pallas_call(
        _tc_body,
        grid=(_GRID,),
        in_specs=[
            pl.BlockSpec((_B, _NF), lambda i: (0, 0)),
            pl.BlockSpec((_BK, _NF), lambda i: (i, 0)),
            pl.BlockSpec((_B, _NF), lambda i: (0, 0)),
        ],
        out_specs=pl.BlockSpec((8, 128), lambda i: (0, 0)),
        out_shape=jax.ShapeDtypeStruct((8, 128), jnp.float32),
        scratch_shapes=[
            pltpu.VMEM((_B, 1), jnp.float32),
            pltpu.VMEM((_B, 1), jnp.float32),
        ],
        compiler_params=pltpu.CompilerParams(
            dimension_semantics=("arbitrary",)),
    )(inputs, fpad, gathered)


@functools.cache
def _make_sc_gather():
    info = plsc.get_sparse_core_info()
    nc, ns = info.num_cores, info.num_subcores
    nw = nc * ns
    b_per_w = _B // nw
    mesh = plsc.VectorSubcoreMesh(core_axis_name="c", subcore_axis_name="s")

    @functools.partial(
        pl.kernel, mesh=mesh,
        out_type=jax.ShapeDtypeStruct((_B, _NF), jnp.float32),
        scratch_types=[
            pltpu.VMEM((b_per_w,), jnp.int32),
            pltpu.VMEM((b_per_w, _NF), jnp.float32),
            pltpu.SemaphoreType.DMA,
        ],
        compiler_params=pltpu.CompilerParams(use_tc_tiling_on_sc=False),
    )
    def gather(table_hbm, idx_hbm, out_hbm, idx_v, rows_v, sem):
        wid = lax.axis_index("s") * nc + lax.axis_index("c")
        base = wid * b_per_w
        pltpu.sync_copy(idx_hbm.at[pl.ds(base, b_per_w)], idx_v)
        pltpu.async_copy(table_hbm.at[idx_v], rows_v, sem).wait()
        pltpu.sync_copy(rows_v, out_hbm.at[pl.ds(base, b_per_w)])

    return gather


def kernel(inputs, targets, features):
    idx = targets.astype(jnp.int32)
    gathered = _make_sc_gather()(features, idx)
    out = _lse_loss(inputs, features, gathered)
    return out[0, 0]


# speedup vs baseline: 3.4295x; 1.3114x over previous
import functools

import jax
import jax.numpy as jnp
from jax import lax
from jax.experimental import pallas as pl
from jax.experimental.pallas import tpu as pltpu
from jax.experimental.pallas import tpu_sc as plsc

_NF = 32
_NCLS = 100000
_B = 1024
_TEMP = 0.05
_BK = 8192
_GRID = (_NCLS + _BK - 1) // _BK
_NPAD = _GRID * _BK


_LOG2E = 1.4426950408889634
_SCALE = _LOG2E / _TEMP
_LN2 = 0.6931471805599453


def _tc_body(x_ref, f_ref, g_ref, out_ref, m_ref, s_ref):
    pid = pl.program_id(0)

    xs = x_ref[...] * _SCALE

    @pl.when(pid == 0)
    def _init():
        m_ref[...] = jnp.sqrt(jnp.sum(xs * xs, axis=1, keepdims=True))
        s_ref[...] = jnp.zeros((_B, 1), jnp.float32)

    def _dot():
        return lax.dot_general(
            xs, f_ref[...], (((1,), (1,)), ((), ())),
            preferred_element_type=jnp.float32,
            precision=lax.Precision.DEFAULT,
        )

    def _update(b):
        c = m_ref[...]
        s_old = s_ref[...]
        raw = jnp.sum(jnp.exp2(b - c), axis=1, keepdims=True)
        good = (jnp.min(raw) > 0.0) & (jnp.max(raw) < 3.0e38)

        @pl.when(good)
        def _fast():
            m_new = jnp.maximum(c, c + jnp.log2(raw))
            s_ref[...] = (s_old + raw) * jnp.exp2(c - m_new)
            m_ref[...] = m_new

        @pl.when(jnp.logical_not(good))
        def _slow():
            bm = jnp.max(b, axis=1, keepdims=True)
            nonzero = s_old > 0.0
            m_new = jnp.maximum(jnp.where(nonzero, c, bm), bm)
            resc = jnp.where(nonzero, jnp.exp2(c - m_new), 0.0)
            s_ref[...] = s_old * resc + jnp.sum(
                jnp.exp2(b - m_new), axis=1, keepdims=True)
            m_ref[...] = m_new

    @pl.when(pid != _GRID - 1)
    def _full():
        _update(_dot())

    @pl.when(pid == _GRID - 1)
    def _tail():
        col = lax.broadcasted_iota(jnp.int32, (1, _BK), 1)
        _update(jnp.where(col < _NCLS - (_GRID - 1) * _BK, _dot(), -1e30))
        lse2 = m_ref[...] + jnp.log2(s_ref[...])
        tgt2 = jnp.sum(xs * g_ref[...], axis=1, keepdims=True)
        loss = jnp.sum(lse2 - tgt2) * (_LN2 / _B)
        out_ref[...] = jnp.full((8, 128), loss, jnp.float32)


def _lse_loss(inputs, fpad, gathered):
    return pl.pallas_call(
        _tc_body,
        grid=(_GRID,),
        in_specs=[
            pl.BlockSpec((_B, _NF), lambda i: (0, 0)),
            pl.BlockSpec((_BK, _NF), lambda i: (i, 0)),
            pl.BlockSpec((_B, _NF), lambda i: (0, 0)),
        ],
        out_specs=pl.BlockSpec((8, 128), lambda i: (0, 0)),
        out_shape=jax.ShapeDtypeStruct((8, 128), jnp.float32),
        scratch_shapes=[
            pltpu.VMEM((_B, 1), jnp.float32),
            pltpu.VMEM((_B, 1), jnp.float32),
        ],
        compiler_params=pltpu.CompilerParams(
            dimension_semantics=("arbitrary",)),
    )(inputs, fpad, gathered)


@functools.cache
def _make_sc_gather():
    info = plsc.get_sparse_core_info()
    nc, ns = info.num_cores, info.num_subcores
    nw = nc * ns
    b_per_w = _B // nw
    mesh = plsc.VectorSubcoreMesh(core_axis_name="c", subcore_axis_name="s")

    @functools.partial(
        pl.kernel, mesh=mesh,
        out_type=jax.ShapeDtypeStruct((_B, _NF), jnp.float32),
        scratch_types=[
            pltpu.VMEM((b_per_w,), jnp.int32),
            pltpu.VMEM((b_per_w, _NF), jnp.float32),
            pltpu.SemaphoreType.DMA,
        ],
        compiler_params=pltpu.CompilerParams(use_tc_tiling_on_sc=False),
    )
    def gather(table_hbm, idx_hbm, out_hbm, idx_v, rows_v, sem):
        wid = lax.axis_index("s") * nc + lax.axis_index("c")
        base = wid * b_per_w
        pltpu.sync_copy(idx_hbm.at[pl.ds(base, b_per_w)], idx_v)
        pltpu.async_copy(table_hbm.at[idx_v], rows_v, sem).wait()
        pltpu.sync_copy(rows_v, out_hbm.at[pl.ds(base, b_per_w)])

    return gather


def kernel(inputs, targets, features):
    idx = targets.astype(jnp.int32)
    gathered = _make_sc_gather()(features, idx)
    out = _lse_loss(inputs, features, gathered)
    return out[0, 0]
